# baseline (speedup 1.0000x reference)
import functools

import jax
import jax.numpy as jnp
from jax import lax
from jax.experimental import pallas as pl
from jax.experimental.pallas import tpu as pltpu
from jax.experimental.pallas import tpu_sc as plsc

_MARGIN = 1.0
_N_NODES = 10000
_D_FEAT = 256
_N_PAIRS = 160000
_LANES = 16
_NS = 16
_PAIRS_PER_TILE = _N_PAIRS // _NS
_VECS_PER_TILE = _PAIRS_PER_TILE // _LANES


_ROW_BLK = 1000


def _rowsum_body(x_ref, o_ref):
    o_ref[...] = jnp.sum(x_ref[...], axis=1, keepdims=True)


def _rowsum(emb):
    return pl.pallas_call(
        _rowsum_body,
        grid=(_N_NODES // _ROW_BLK,),
        in_specs=[pl.BlockSpec((_ROW_BLK, _D_FEAT), lambda i: (i, 0))],
        out_specs=pl.BlockSpec((_ROW_BLK, 1), lambda i: (i, 0)),
        out_shape=jax.ShapeDtypeStruct((_N_NODES, 1), jnp.float32),
    )(emb)


_sc_mesh = plsc.VectorSubcoreMesh(
    core_axis_name="c", subcore_axis_name="s", num_cores=2, num_subcores=_NS)


@functools.partial(
    pl.kernel,
    out_type=jax.ShapeDtypeStruct((_LANES,), jnp.float32),
    mesh=_sc_mesh,
    compiler_params=pltpu.CompilerParams(needs_layout_passes=False),
    scratch_types=[
        pltpu.VMEM((_N_NODES,), jnp.float32),
        pltpu.VMEM((2 * _PAIRS_PER_TILE,), jnp.int32),
        pltpu.VMEM((_LANES,), jnp.int32),
        pltpu.VMEM((_LANES,), jnp.int32),
        pltpu.VMEM((_LANES,), jnp.float32),
        pltpu.VMEM((_NS * _LANES,), jnp.float32),
        pltpu.VMEM_SHARED((_NS * _LANES,), jnp.float32),
        pltpu.SemaphoreType.DMA,
        pltpu.SemaphoreType.DMA,
    ],
)
def _sc_loss(s_hbm, pairs_hbm, na_hbm, nb_hbm, out_hbm,
             s_v, pr_v, na_v, nb_v, res_v, parts_v, parts_sh, sem0, sem1):
    c = lax.axis_index("c")
    sid = lax.axis_index("s")

    @pl.when(c == 0)
    def _work():
        base = sid * (2 * _PAIRS_PER_TILE)
        cp_s = pltpu.async_copy(s_hbm, s_v, sem0)
        cp_p = pltpu.async_copy(
            pairs_hbm.at[pl.ds(base, 2 * _PAIRS_PER_TILE)], pr_v, sem1)
        cp_s.wait()
        cp_p.wait()

        lane = lax.iota(jnp.int32, _LANES)
        gidx_e = (lane * 2) % _LANES
        low = lane < (_LANES // 2)

        def body(i, acc):
            v0 = pr_v[pl.ds(i * (2 * _LANES), _LANES)]
            v1 = pr_v[pl.ds(i * (2 * _LANES) + _LANES, _LANES)]
            ip = jnp.where(low, jnp.take(v0, gidx_e), jnp.take(v1, gidx_e))
            ic = jnp.where(low, jnp.take(v0, gidx_e + 1),
                           jnp.take(v1, gidx_e + 1))
            gp = plsc.load_gather(s_v, [ip])
            gc = plsc.load_gather(s_v, [ic])
            return acc + jnp.maximum(_MARGIN - gp + gc, 0.0)

        acc = lax.fori_loop(0, _VECS_PER_TILE, body,
                            jnp.zeros((_LANES,), jnp.float32))
        res_v[...] = acc
        pltpu.sync_copy(res_v, parts_sh.at[pl.ds(sid * _LANES, _LANES)])

    plsc.subcore_barrier()

    @pl.when((c == 0) & (sid == 0))
    def _finalize():
        cp_a = pltpu.async_copy(na_hbm, na_v, sem0)
        cp_b = pltpu.async_copy(nb_hbm, nb_v, sem1)
        pltpu.sync_copy(parts_sh, parts_v)
        tot = jnp.zeros((_LANES,), jnp.float32)
        for w in range(_NS):
            tot = tot + parts_v[pl.ds(w * _LANES, _LANES)]
        pos_loss = jnp.sum(tot)

        cp_a.wait()
        cp_b.wait()
        ia = na_v[...]
        ib = nb_v[...]
        ga = plsc.load_gather(s_v, [ia])
        gb = plsc.load_gather(s_v, [ib])
        d12 = ga - gb
        neg = jnp.maximum(d12 - _MARGIN, 0.0) + jnp.maximum(-d12 - _MARGIN, 0.0)
        valid = jnp.where(ia != ib, 1.0, 0.0).astype(jnp.float32)
        neg_loss = jnp.sum(neg * valid)
        vcnt = jnp.sum(valid)

        numer = jnp.full((_LANES,), pos_loss + neg_loss, jnp.float32)
        denom = jnp.full((_LANES,), jnp.float32(_N_PAIRS) + vcnt, jnp.float32)
        res_v[...] = numer / denom
        pltpu.sync_copy(res_v, out_hbm)


def kernel(node_embeddings, parent_child_pairs, neg_idx):
    s = _rowsum(node_embeddings).reshape(_N_NODES)
    pairs_flat = parent_child_pairs.reshape(2 * _N_PAIRS)
    n_neg = neg_idx.shape[0]
    na = jnp.pad(neg_idx[:, 0], (0, _LANES - n_neg))
    nb = jnp.pad(neg_idx[:, 1], (0, _LANES - n_neg))
    out = _sc_loss(s, pairs_flat, na, nb)
    return out[0]


# speedup vs baseline: 2.9109x; 2.9109x over previous
import functools

import jax
import jax.numpy as jnp
from jax import lax
from jax.experimental import pallas as pl
from jax.experimental.pallas import tpu as pltpu
from jax.experimental.pallas import tpu_sc as plsc

_MARGIN = 1.0
_N_NODES = 10000
_D_FEAT = 256
_N_PAIRS = 160000
_LANES = 16
_NS = 16
_PAIRS_PER_TILE = _N_PAIRS // _NS
_VECS_PER_TILE = _PAIRS_PER_TILE // _LANES


_ROW_BLK = 1000


def _rowsum_body(x_ref, o_ref):
    o_ref[...] = jnp.sum(x_ref[...], axis=1, keepdims=True)


def _rowsum(emb):
    return pl.pallas_call(
        _rowsum_body,
        grid=(_N_NODES // _ROW_BLK,),
        in_specs=[pl.BlockSpec((_ROW_BLK, _D_FEAT), lambda i: (i, 0))],
        out_specs=pl.BlockSpec((_ROW_BLK, 1), lambda i: (i, 0)),
        out_shape=jax.ShapeDtypeStruct((_N_NODES, 1), jnp.float32),
    )(emb)


_sc_mesh = plsc.VectorSubcoreMesh(
    core_axis_name="c", subcore_axis_name="s", num_cores=2, num_subcores=_NS)


@functools.partial(
    pl.kernel,
    out_type=jax.ShapeDtypeStruct((_LANES,), jnp.float32),
    mesh=_sc_mesh,
    compiler_params=pltpu.CompilerParams(needs_layout_passes=False),
    scratch_types=[
        pltpu.VMEM((_N_NODES,), jnp.float32),
        pltpu.VMEM((_PAIRS_PER_TILE,), jnp.int32),
        pltpu.VMEM((_LANES,), jnp.int32),
        pltpu.VMEM((_LANES,), jnp.int32),
        pltpu.VMEM((_LANES,), jnp.float32),
        pltpu.VMEM((_NS * _LANES,), jnp.float32),
        pltpu.VMEM_SHARED((_NS * _LANES,), jnp.float32),
        pltpu.SemaphoreType.DMA,
        pltpu.SemaphoreType.DMA,
    ],
)
def _sc_loss(s_hbm, packed_hbm, na_hbm, nb_hbm, out_hbm,
             s_v, pk_v, na_v, nb_v, res_v, parts_v, parts_sh, sem0, sem1):
    c = lax.axis_index("c")
    sid = lax.axis_index("s")

    @pl.when(c == 0)
    def _work():
        base = sid * _PAIRS_PER_TILE
        cp_s = pltpu.async_copy(s_hbm, s_v, sem0)
        cp_p = pltpu.async_copy(
            packed_hbm.at[pl.ds(base, _PAIRS_PER_TILE)], pk_v, sem1)
        cp_s.wait()
        cp_p.wait()

        def body(i, acc):
            pk = pk_v[pl.ds(i * _LANES, _LANES)]
            ip = lax.shift_right_logical(pk, 16)
            ic = lax.bitwise_and(pk, 0xFFFF)
            gp = plsc.load_gather(s_v, [ip])
            gc = plsc.load_gather(s_v, [ic])
            return acc + jnp.maximum(_MARGIN - gp + gc, 0.0)

        acc = lax.fori_loop(0, _VECS_PER_TILE, body,
                            jnp.zeros((_LANES,), jnp.float32))
        res_v[...] = acc
        pltpu.sync_copy(res_v, parts_sh.at[pl.ds(sid * _LANES, _LANES)])

    plsc.subcore_barrier()

    @pl.when((c == 0) & (sid == 0))
    def _finalize():
        cp_a = pltpu.async_copy(na_hbm, na_v, sem0)
        cp_b = pltpu.async_copy(nb_hbm, nb_v, sem1)
        pltpu.sync_copy(parts_sh, parts_v)
        tot = jnp.zeros((_LANES,), jnp.float32)
        for w in range(_NS):
            tot = tot + parts_v[pl.ds(w * _LANES, _LANES)]
        pos_loss = jnp.sum(tot)

        cp_a.wait()
        cp_b.wait()
        ia = na_v[...]
        ib = nb_v[...]
        ga = plsc.load_gather(s_v, [ia])
        gb = plsc.load_gather(s_v, [ib])
        d12 = ga - gb
        neg = jnp.maximum(d12 - _MARGIN, 0.0) + jnp.maximum(-d12 - _MARGIN, 0.0)
        valid = jnp.where(ia != ib, 1.0, 0.0).astype(jnp.float32)
        neg_loss = jnp.sum(neg * valid)
        vcnt = jnp.sum(valid)

        numer = jnp.full((_LANES,), pos_loss + neg_loss, jnp.float32)
        denom = jnp.full((_LANES,), jnp.float32(_N_PAIRS) + vcnt, jnp.float32)
        res_v[...] = numer / denom
        pltpu.sync_copy(res_v, out_hbm)


def kernel(node_embeddings, parent_child_pairs, neg_idx):
    s = _rowsum(node_embeddings).reshape(_N_NODES)
    packed = parent_child_pairs[:, 0] * 65536 + parent_child_pairs[:, 1]
    n_neg = neg_idx.shape[0]
    na = jnp.pad(neg_idx[:, 0], (0, _LANES - n_neg))
    nb = jnp.pad(neg_idx[:, 1], (0, _LANES - n_neg))
    out = _sc_loss(s, packed, na, nb)
    return out[0]


# speedup vs baseline: 2.9549x; 1.0151x over previous
import functools

import jax
import jax.numpy as jnp
from jax import lax
from jax.experimental import pallas as pl
from jax.experimental.pallas import tpu as pltpu
from jax.experimental.pallas import tpu_sc as plsc

_MARGIN = 1.0
_N_NODES = 10000
_D_FEAT = 256
_N_PAIRS = 160000
_LANES = 16
_NS = 16
_PAIRS_PER_TILE = _N_PAIRS // _NS
_VECS_PER_TILE = _PAIRS_PER_TILE // _LANES


_ROW_BLK = 1000


def _rowsum_body(x_ref, o_ref):
    o_ref[...] = jnp.sum(x_ref[...], axis=1, keepdims=True)


def _rowsum(emb):
    return pl.pallas_call(
        _rowsum_body,
        grid=(_N_NODES // _ROW_BLK,),
        in_specs=[pl.BlockSpec((_ROW_BLK, _D_FEAT), lambda i: (i, 0))],
        out_specs=pl.BlockSpec((_ROW_BLK, 1), lambda i: (i, 0)),
        out_shape=jax.ShapeDtypeStruct((_N_NODES, 1), jnp.float32),
    )(emb)


_sc_mesh = plsc.VectorSubcoreMesh(
    core_axis_name="c", subcore_axis_name="s", num_cores=2, num_subcores=_NS)


@functools.partial(
    pl.kernel,
    out_type=jax.ShapeDtypeStruct((_LANES,), jnp.float32),
    mesh=_sc_mesh,
    compiler_params=pltpu.CompilerParams(needs_layout_passes=False),
    scratch_types=[
        pltpu.VMEM((_N_NODES,), jnp.float32),
        pltpu.VMEM((_PAIRS_PER_TILE,), jnp.int32),
        pltpu.VMEM((_LANES,), jnp.int32),
        pltpu.VMEM((_LANES,), jnp.float32),
        pltpu.VMEM((_NS * _LANES,), jnp.float32),
        pltpu.VMEM_SHARED((_NS * _LANES,), jnp.float32),
        pltpu.SemaphoreType.DMA,
        pltpu.SemaphoreType.DMA,
    ],
)
def _sc_loss(s_hbm, packed_hbm, out_hbm,
             s_v, pk_v, ng_v, res_v, parts_v, parts_sh, sem0, sem1):
    c = lax.axis_index("c")
    sid = lax.axis_index("s")

    @pl.when(c == 0)
    def _work():
        base = sid * _PAIRS_PER_TILE
        cp_s = pltpu.async_copy(s_hbm, s_v, sem0)
        cp_p = pltpu.async_copy(
            packed_hbm.at[pl.ds(base, _PAIRS_PER_TILE)], pk_v, sem1)
        cp_s.wait()
        cp_p.wait()

        def body(i, acc):
            pk = pk_v[pl.ds(i * _LANES, _LANES)]
            ip = lax.shift_right_logical(pk, 16)
            ic = lax.bitwise_and(pk, 0xFFFF)
            gp = plsc.load_gather(s_v, [ip])
            gc = plsc.load_gather(s_v, [ic])
            return acc + jnp.maximum(_MARGIN - gp + gc, 0.0)

        acc = lax.fori_loop(0, _VECS_PER_TILE, body,
                            jnp.zeros((_LANES,), jnp.float32))
        res_v[...] = acc
        pltpu.sync_copy(res_v, parts_sh.at[pl.ds(sid * _LANES, _LANES)])

    plsc.subcore_barrier()

    @pl.when((c == 0) & (sid == 0))
    def _finalize():
        cp_n = pltpu.async_copy(
            packed_hbm.at[pl.ds(_N_PAIRS, _LANES)], ng_v, sem0)
        pltpu.sync_copy(parts_sh, parts_v)
        tot = jnp.zeros((_LANES,), jnp.float32)
        for w in range(_NS):
            tot = tot + parts_v[pl.ds(w * _LANES, _LANES)]
        pos_loss = jnp.sum(tot)

        cp_n.wait()
        pkn = ng_v[...]
        ia = lax.shift_right_logical(pkn, 16)
        ib = lax.bitwise_and(pkn, 0xFFFF)
        ga = plsc.load_gather(s_v, [ia])
        gb = plsc.load_gather(s_v, [ib])
        d12 = ga - gb
        neg = jnp.maximum(d12 - _MARGIN, 0.0) + jnp.maximum(-d12 - _MARGIN, 0.0)
        valid = jnp.where(ia != ib, 1.0, 0.0).astype(jnp.float32)
        neg_loss = jnp.sum(neg * valid)
        vcnt = jnp.sum(valid)

        numer = jnp.full((_LANES,), pos_loss + neg_loss, jnp.float32)
        denom = jnp.full((_LANES,), jnp.float32(_N_PAIRS) + vcnt, jnp.float32)
        res_v[...] = numer / denom
        pltpu.sync_copy(res_v, out_hbm)


def kernel(node_embeddings, parent_child_pairs, neg_idx):
    s = _rowsum(node_embeddings).reshape(_N_NODES)
    packed = parent_child_pairs[:, 0] * 65536 + parent_child_pairs[:, 1]
    pneg = neg_idx[:, 0] * 65536 + neg_idx[:, 1]
    n_neg = neg_idx.shape[0]
    packed_all = jnp.concatenate(
        [packed, jnp.pad(pneg, (0, _LANES - n_neg))])
    out = _sc_loss(s, packed_all)
    return out[0]


# speedup vs baseline: 3.0251x; 1.0237x over previous
import functools

import jax
import jax.numpy as jnp
from jax import lax
from jax.experimental import pallas as pl
from jax.experimental.pallas import tpu as pltpu
from jax.experimental.pallas import tpu_sc as plsc

_MARGIN = 1.0
_N_NODES = 10000
_D_FEAT = 256
_N_PAIRS = 160000
_LANES = 16
_NS = 16
_PAIRS_PER_TILE = _N_PAIRS // _NS
_VECS_PER_TILE = _PAIRS_PER_TILE // _LANES
_UNROLL = 5


_ROW_BLK = 1000


def _rowsum_body(x_ref, o_ref):
    o_ref[...] = jnp.sum(x_ref[...], axis=1, keepdims=True)


def _rowsum(emb):
    return pl.pallas_call(
        _rowsum_body,
        grid=(_N_NODES // _ROW_BLK,),
        in_specs=[pl.BlockSpec((_ROW_BLK, _D_FEAT), lambda i: (i, 0))],
        out_specs=pl.BlockSpec((_ROW_BLK, 1), lambda i: (i, 0)),
        out_shape=jax.ShapeDtypeStruct((_N_NODES, 1), jnp.float32),
    )(emb)


_sc_mesh = plsc.VectorSubcoreMesh(
    core_axis_name="c", subcore_axis_name="s", num_cores=2, num_subcores=_NS)


@functools.partial(
    pl.kernel,
    out_type=jax.ShapeDtypeStruct((_LANES,), jnp.float32),
    mesh=_sc_mesh,
    compiler_params=pltpu.CompilerParams(needs_layout_passes=False),
    scratch_types=[
        pltpu.VMEM((_N_NODES,), jnp.float32),
        pltpu.VMEM((_PAIRS_PER_TILE,), jnp.int32),
        pltpu.VMEM((_LANES,), jnp.int32),
        pltpu.VMEM((_LANES,), jnp.float32),
        pltpu.VMEM((_NS * _LANES,), jnp.float32),
        pltpu.VMEM_SHARED((_NS * _LANES,), jnp.float32),
        pltpu.SemaphoreType.DMA,
        pltpu.SemaphoreType.DMA,
    ],
)
def _sc_loss(s_hbm, packed_hbm, out_hbm,
             s_v, pk_v, ng_v, res_v, parts_v, parts_sh, sem0, sem1):
    c = lax.axis_index("c")
    sid = lax.axis_index("s")

    @pl.when(c == 0)
    def _work():
        base = sid * _PAIRS_PER_TILE
        cp_s = pltpu.async_copy(s_hbm, s_v, sem0)
        cp_p = pltpu.async_copy(
            packed_hbm.at[pl.ds(base, _PAIRS_PER_TILE)], pk_v, sem1)
        cp_s.wait()
        cp_p.wait()

        def body(i, acc):
            for k in range(_UNROLL):
                pk = pk_v[pl.ds(i * (_UNROLL * _LANES) + k * _LANES, _LANES)]
                ip = lax.shift_right_logical(pk, 16)
                ic = lax.bitwise_and(pk, 0xFFFF)
                gp = plsc.load_gather(s_v, [ip])
                gc = plsc.load_gather(s_v, [ic])
                acc = acc + jnp.maximum(_MARGIN - gp + gc, 0.0)
            return acc

        acc = lax.fori_loop(0, _VECS_PER_TILE // _UNROLL, body,
                            jnp.zeros((_LANES,), jnp.float32))
        res_v[...] = acc
        pltpu.sync_copy(res_v, parts_sh.at[pl.ds(sid * _LANES, _LANES)])

    plsc.subcore_barrier()

    @pl.when((c == 0) & (sid == 0))
    def _finalize():
        cp_n = pltpu.async_copy(
            packed_hbm.at[pl.ds(_N_PAIRS, _LANES)], ng_v, sem0)
        pltpu.sync_copy(parts_sh, parts_v)
        tot = jnp.zeros((_LANES,), jnp.float32)
        for w in range(_NS):
            tot = tot + parts_v[pl.ds(w * _LANES, _LANES)]
        pos_loss = jnp.sum(tot)

        cp_n.wait()
        pkn = ng_v[...]
        ia = lax.shift_right_logical(pkn, 16)
        ib = lax.bitwise_and(pkn, 0xFFFF)
        ga = plsc.load_gather(s_v, [ia])
        gb = plsc.load_gather(s_v, [ib])
        d12 = ga - gb
        neg = jnp.maximum(d12 - _MARGIN, 0.0) + jnp.maximum(-d12 - _MARGIN, 0.0)
        valid = jnp.where(ia != ib, 1.0, 0.0).astype(jnp.float32)
        neg_loss = jnp.sum(neg * valid)
        vcnt = jnp.sum(valid)

        numer = jnp.full((_LANES,), pos_loss + neg_loss, jnp.float32)
        denom = jnp.full((_LANES,), jnp.float32(_N_PAIRS) + vcnt, jnp.float32)
        res_v[...] = numer / denom
        pltpu.sync_copy(res_v, out_hbm)


def kernel(node_embeddings, parent_child_pairs, neg_idx):
    s = _rowsum(node_embeddings).reshape(_N_NODES)
    packed = parent_child_pairs[:, 0] * 65536 + parent_child_pairs[:, 1]
    pneg = neg_idx[:, 0] * 65536 + neg_idx[:, 1]
    n_neg = neg_idx.shape[0]
    packed_all = jnp.concatenate(
        [packed, jnp.pad(pneg, (0, _LANES - n_neg))])
    out = _sc_loss(s, packed_all)
    return out[0]
